# baseline (speedup 1.0000x reference)
import jax
import jax.numpy as jnp
from jax import lax
from jax.experimental import pallas as pl
from jax.experimental.pallas import tpu as pltpu
from jax.experimental.pallas import tpu_sc as plsc

VOCAB = 1000000
DIM = 16
BATCH = 16384

NC = 2
NS = 16
NW = NC * NS
LANES = 16

B_PER_W = BATCH // NW
CHUNK = 128
N_CHUNKS = B_PER_W // CHUNK
GROUPS = CHUNK // LANES

LANE_TILES = (VOCAB + 127) // 128
PLANE_WORDS = LANE_TILES * 1024
FLAT = DIM * VOCAB


def _sc_body(u_hbm, v_hbm, table_hbm, out_hbm,
             idx_u, idx_v, off_u, off_v, ubuf, vbuf, out_loc, sem):
    wid = lax.axis_index("s") * NC + lax.axis_index("c")
    base = wid * B_PER_W
    for c in range(N_CHUNKS):
        pltpu.sync_copy(u_hbm.at[pl.ds(base + c * CHUNK, CHUNK)], idx_u.at[c])
        pltpu.sync_copy(v_hbm.at[pl.ds(base + c * CHUNK, CHUNK)], idx_v.at[c])

    def build_offsets(c, b):
        def slice_fn(g, _):
            o = g * LANES
            ru = idx_u.at[c][pl.ds(o, LANES)]
            rv = idx_v.at[c][pl.ds(o, LANES)]
            for d in range(DIM):
                off_u[b, d, pl.ds(o, LANES)] = ru + d * VOCAB
                off_v[b, d, pl.ds(o, LANES)] = rv + d * VOCAB
            return _

        lax.fori_loop(0, GROUPS, slice_fn, None)

    def fire(b):
        copies = []
        for d in range(DIM):
            copies.append(pltpu.async_copy(
                table_hbm.at[off_u.at[b, d]], ubuf.at[b, d], sem))
            copies.append(pltpu.async_copy(
                table_hbm.at[off_v.at[b, d]], vbuf.at[b, d], sem))
        return copies

    def compute_chunk(c, b):
        def group(g, _):
            o = g * LANES
            acc = jnp.zeros((LANES,), jnp.float32)
            for d in range(DIM):
                acc = acc + ubuf.at[b, d][pl.ds(o, LANES)] * vbuf.at[b, d][pl.ds(o, LANES)]
            out_loc[pl.ds(c * CHUNK + o, LANES)] = 1.0 / (1.0 + jnp.exp(-acc))
            return _

        lax.fori_loop(0, GROUPS, group, None)

    build_offsets(0, 0)
    pending = fire(0)
    for c in range(N_CHUNKS):
        b = c % 2
        if c + 1 < N_CHUNKS:
            build_offsets(c + 1, 1 - b)
        for cp in pending:
            cp.wait()
        if c + 1 < N_CHUNKS:
            nxt = fire(1 - b)
        compute_chunk(c, b)
        if c + 1 < N_CHUNKS:
            pending = nxt

    pltpu.sync_copy(out_loc, out_hbm.at[pl.ds(base, B_PER_W)])


@jax.jit
def kernel(u, v, embed):
    mesh = plsc.VectorSubcoreMesh(
        core_axis_name="c", subcore_axis_name="s",
        num_cores=NC, num_subcores=NS,
    )
    k = pl.kernel(
        _sc_body,
        out_type=jax.ShapeDtypeStruct((BATCH,), jnp.float32),
        mesh=mesh,
        scratch_types=[
            pltpu.VMEM((N_CHUNKS, CHUNK), jnp.int32),
            pltpu.VMEM((N_CHUNKS, CHUNK), jnp.int32),
            pltpu.VMEM((2, DIM, CHUNK), jnp.int32),
            pltpu.VMEM((2, DIM, CHUNK), jnp.int32),
            pltpu.VMEM((2, DIM, CHUNK), jnp.float32),
            pltpu.VMEM((2, DIM, CHUNK), jnp.float32),
            pltpu.VMEM((B_PER_W,), jnp.float32),
            pltpu.SemaphoreType.DMA,
        ],
        compiler_params=pltpu.CompilerParams(
            needs_layout_passes=False, use_tc_tiling_on_sc=False),
    )
    tflat = jnp.reshape(embed.T, (DIM * VOCAB,))
    return k(u.astype(jnp.int32), v.astype(jnp.int32), tflat)


# speedup vs baseline: 10.7645x; 10.7645x over previous
import jax
import jax.numpy as jnp
from jax import lax
from jax.experimental import pallas as pl
from jax.experimental.pallas import tpu as pltpu
from jax.experimental.pallas import tpu_sc as plsc

VOCAB = 1000000
DIM = 16
BATCH = 16384

NC = 2
NS = 16
NW = NC * NS
LANES = 16

B_PER_W = BATCH // NW
CHUNK = 128
N_CHUNKS = B_PER_W // CHUNK
GROUPS = CHUNK // LANES

VPAD = 1048576
VBULK = 999936
VTAIL = VOCAB - VBULK
DETILE_BLK = 65536
FLAT = DIM * VPAD


def _sc_body(u_hbm, v_hbm, table_hbm, tail_hbm, out_hbm,
             idx_u, idx_v, off_u, off_v, ubuf, vbuf, tail_v, out_loc, sem):
    wid = lax.axis_index("s") * NC + lax.axis_index("c")
    base = wid * B_PER_W
    pltpu.sync_copy(tail_hbm, tail_v)
    for c in range(N_CHUNKS):
        pltpu.sync_copy(u_hbm.at[pl.ds(base + c * CHUNK, CHUNK)], idx_u.at[c])
        pltpu.sync_copy(v_hbm.at[pl.ds(base + c * CHUNK, CHUNK)], idx_v.at[c])

    def build_offsets(c, b):
        def slice_fn(g, _):
            o = g * LANES
            ru = idx_u.at[c][pl.ds(o, LANES)]
            rv = idx_v.at[c][pl.ds(o, LANES)]
            ru = jnp.minimum(ru, VBULK - 1)
            rv = jnp.minimum(rv, VBULK - 1)
            for d in range(DIM):
                off_u[b, d, pl.ds(o, LANES)] = ru + d * VPAD
                off_v[b, d, pl.ds(o, LANES)] = rv + d * VPAD
            return _

        lax.fori_loop(0, GROUPS, slice_fn, None)

    def fire(b):
        copies = []
        for d in range(DIM):
            copies.append(pltpu.async_copy(
                table_hbm.at[off_u.at[b, d]], ubuf.at[b, d], sem))
            copies.append(pltpu.async_copy(
                table_hbm.at[off_v.at[b, d]], vbuf.at[b, d], sem))
        return copies

    def compute_chunk(c, b):
        def group(g, _):
            o = g * LANES
            ru = idx_u.at[c][pl.ds(o, LANES)]
            rv = idx_v.at[c][pl.ds(o, LANES)]
            mu = ru >= VBULK
            mv = rv >= VBULK
            tu = jnp.clip(ru - VBULK, 0, VTAIL - 1)
            tv = jnp.clip(rv - VBULK, 0, VTAIL - 1)
            acc = jnp.zeros((LANES,), jnp.float32)
            for d in range(DIM):
                uu = ubuf.at[b, d][pl.ds(o, LANES)]
                vv = vbuf.at[b, d][pl.ds(o, LANES)]
                uu = jnp.where(mu, plsc.load_gather(tail_v, [tu + d * VTAIL]), uu)
                vv = jnp.where(mv, plsc.load_gather(tail_v, [tv + d * VTAIL]), vv)
                acc = acc + uu * vv
            out_loc[pl.ds(c * CHUNK + o, LANES)] = 1.0 / (1.0 + jnp.exp(-acc))
            return _

        lax.fori_loop(0, GROUPS, group, None)

    build_offsets(0, 0)
    pending = fire(0)
    for c in range(N_CHUNKS):
        b = c % 2
        if c + 1 < N_CHUNKS:
            build_offsets(c + 1, 1 - b)
        for cp in pending:
            cp.wait()
        if c + 1 < N_CHUNKS:
            nxt = fire(1 - b)
        compute_chunk(c, b)
        if c + 1 < N_CHUNKS:
            pending = nxt

    pltpu.sync_copy(out_loc, out_hbm.at[pl.ds(base, B_PER_W)])


def _detile_body(in_hbm, out_hbm, buf, sem_in, sem_out):
    steps = []
    for p in range(DIM // 8):
        for k in range((VBULK + DETILE_BLK - 1) // DETILE_BLK):
            n = min(DETILE_BLK, VBULK - k * DETILE_BLK)
            steps.append((p, k, n))

    def fire_in(s, b):
        p, k, n = steps[s]
        cp = pltpu.make_async_copy(
            in_hbm.at[pl.ds(p * 8, 8), pl.ds(k * DETILE_BLK, n)],
            buf.at[b, :, pl.ds(0, n)], sem_in.at[b])
        cp.start()
        return cp

    def fire_out(s, b):
        p, k, n = steps[s]
        cps = []
        for j in range(8):
            cp = pltpu.make_async_copy(
                buf.at[b, j, pl.ds(0, n)],
                out_hbm.at[pl.ds((p * 8 + j) * VPAD + k * DETILE_BLK, n)],
                sem_out.at[b])
            cp.start()
            cps.append(cp)
        return cps

    nsteps = len(steps)
    pending_in = fire_in(0, 0)
    pending_out = None
    for s in range(nsteps):
        b = s % 2
        pending_in.wait()
        if s + 1 < nsteps:
            if pending_out is not None:
                for cp in pending_out:
                    cp.wait()
            nxt = fire_in(s + 1, 1 - b)
        out_cps = fire_out(s, b)
        if s + 1 < nsteps:
            pending_out, pending_in = out_cps, nxt
        else:
            if pending_out is not None:
                for cp in pending_out:
                    cp.wait()
            for cp in out_cps:
                cp.wait()


def _detile(table_t):
    return pl.pallas_call(
        _detile_body,
        in_specs=[pl.BlockSpec(memory_space=pl.ANY)],
        out_specs=pl.BlockSpec(memory_space=pl.ANY),
        out_shape=jax.ShapeDtypeStruct((FLAT,), jnp.float32),
        scratch_shapes=[
            pltpu.VMEM((2, 8, DETILE_BLK), jnp.float32),
            pltpu.SemaphoreType.DMA((2,)),
            pltpu.SemaphoreType.DMA((2,)),
        ],
    )(table_t)


@jax.jit
def kernel(u, v, embed):
    mesh = plsc.VectorSubcoreMesh(
        core_axis_name="c", subcore_axis_name="s",
        num_cores=NC, num_subcores=NS,
    )
    k = pl.kernel(
        _sc_body,
        out_type=jax.ShapeDtypeStruct((BATCH,), jnp.float32),
        mesh=mesh,
        scratch_types=[
            pltpu.VMEM((N_CHUNKS, CHUNK), jnp.int32),
            pltpu.VMEM((N_CHUNKS, CHUNK), jnp.int32),
            pltpu.VMEM((2, DIM, CHUNK), jnp.int32),
            pltpu.VMEM((2, DIM, CHUNK), jnp.int32),
            pltpu.VMEM((2, DIM, CHUNK), jnp.float32),
            pltpu.VMEM((2, DIM, CHUNK), jnp.float32),
            pltpu.VMEM((DIM * VTAIL,), jnp.float32),
            pltpu.VMEM((B_PER_W,), jnp.float32),
            pltpu.SemaphoreType.DMA,
        ],
        compiler_params=pltpu.CompilerParams(
            needs_layout_passes=False, use_tc_tiling_on_sc=False),
    )
    tflat = _detile(embed.T)
    tail = jnp.reshape(embed[VBULK:].T, (DIM * VTAIL,))
    return k(u.astype(jnp.int32), v.astype(jnp.int32), tflat, tail)


# speedup vs baseline: 12.4059x; 1.1525x over previous
import jax
import jax.numpy as jnp
from jax import lax
from jax.experimental import pallas as pl
from jax.experimental.pallas import tpu as pltpu
from jax.experimental.pallas import tpu_sc as plsc

VOCAB = 1000000
DIM = 16
BATCH = 16384

NC = 2
NS = 16
NW = NC * NS
LANES = 16

B_PER_W = BATCH // NW
CHUNK = 128
N_CHUNKS = B_PER_W // CHUNK
GROUPS = CHUNK // LANES

VPAD = 1048576
VBULK = 999936
VTAIL = VOCAB - VBULK
DETILE_BLK = 131072
FLAT = DIM * VPAD


def _sc_body(u_hbm, v_hbm, table_hbm, tail_hbm, out_hbm,
             idx_u, idx_v, off_u, off_v, ubuf, vbuf, tail_v, out_loc, sem):
    wid = lax.axis_index("s") * NC + lax.axis_index("c")
    base = wid * B_PER_W
    pltpu.sync_copy(tail_hbm, tail_v)
    for c in range(N_CHUNKS):
        pltpu.sync_copy(u_hbm.at[pl.ds(base + c * CHUNK, CHUNK)], idx_u.at[c])
        pltpu.sync_copy(v_hbm.at[pl.ds(base + c * CHUNK, CHUNK)], idx_v.at[c])

    def build_offsets(c, b):
        def slice_fn(g, _):
            o = g * LANES
            ru = idx_u.at[c][pl.ds(o, LANES)]
            rv = idx_v.at[c][pl.ds(o, LANES)]
            ru = jnp.minimum(ru, VBULK - 1)
            rv = jnp.minimum(rv, VBULK - 1)
            for d in range(DIM):
                off_u[b, d, pl.ds(o, LANES)] = ru + d * VPAD
                off_v[b, d, pl.ds(o, LANES)] = rv + d * VPAD
            return _

        lax.fori_loop(0, GROUPS, slice_fn, None)

    def fire(b):
        copies = []
        for d in range(DIM):
            copies.append(pltpu.async_copy(
                table_hbm.at[off_u.at[b, d]], ubuf.at[b, d], sem))
            copies.append(pltpu.async_copy(
                table_hbm.at[off_v.at[b, d]], vbuf.at[b, d], sem))
        return copies

    def compute_chunk(c, b):
        def group(g, _):
            o = g * LANES
            ru = idx_u.at[c][pl.ds(o, LANES)]
            rv = idx_v.at[c][pl.ds(o, LANES)]
            mu = ru >= VBULK
            mv = rv >= VBULK
            tu = jnp.clip(ru - VBULK, 0, VTAIL - 1)
            tv = jnp.clip(rv - VBULK, 0, VTAIL - 1)
            acc = jnp.zeros((LANES,), jnp.float32)
            for d in range(DIM):
                uu = ubuf.at[b, d][pl.ds(o, LANES)]
                vv = vbuf.at[b, d][pl.ds(o, LANES)]
                uu = jnp.where(mu, plsc.load_gather(tail_v, [tu + d * VTAIL]), uu)
                vv = jnp.where(mv, plsc.load_gather(tail_v, [tv + d * VTAIL]), vv)
                acc = acc + uu * vv
            out_loc[pl.ds(c * CHUNK + o, LANES)] = 1.0 / (1.0 + jnp.exp(-acc))
            return _

        lax.fori_loop(0, GROUPS, group, None)

    build_offsets(0, 0)
    pending = fire(0)
    for c in range(N_CHUNKS):
        b = c % 2
        if c + 1 < N_CHUNKS:
            build_offsets(c + 1, 1 - b)
        for cp in pending:
            cp.wait()
        if c + 1 < N_CHUNKS:
            nxt = fire(1 - b)
        compute_chunk(c, b)
        if c + 1 < N_CHUNKS:
            pending = nxt

    pltpu.sync_copy(out_loc, out_hbm.at[pl.ds(base, B_PER_W)])


def _detile_body(in_hbm, out_hbm, buf, sem_in, sem_out):
    steps = []
    for p in range(DIM // 8):
        for k in range((VBULK + DETILE_BLK - 1) // DETILE_BLK):
            n = min(DETILE_BLK, VBULK - k * DETILE_BLK)
            steps.append((p, k, n))

    def fire_in(s, b):
        p, k, n = steps[s]
        cp = pltpu.make_async_copy(
            in_hbm.at[pl.ds(p * 8, 8), pl.ds(k * DETILE_BLK, n)],
            buf.at[b, :, pl.ds(0, n)], sem_in.at[b])
        cp.start()
        return cp

    def fire_out(s, b):
        p, k, n = steps[s]
        cps = []
        for j in range(8):
            cp = pltpu.make_async_copy(
                buf.at[b, j, pl.ds(0, n)],
                out_hbm.at[pl.ds((p * 8 + j) * VPAD + k * DETILE_BLK, n)],
                sem_out.at[b])
            cp.start()
            cps.append(cp)
        return cps

    nsteps = len(steps)
    pending_in = fire_in(0, 0)
    pending_out = None
    for s in range(nsteps):
        b = s % 2
        pending_in.wait()
        if s + 1 < nsteps:
            if pending_out is not None:
                for cp in pending_out:
                    cp.wait()
            nxt = fire_in(s + 1, 1 - b)
        out_cps = fire_out(s, b)
        if s + 1 < nsteps:
            pending_out, pending_in = out_cps, nxt
        else:
            if pending_out is not None:
                for cp in pending_out:
                    cp.wait()
            for cp in out_cps:
                cp.wait()


def _detile(table_t):
    return pl.pallas_call(
        _detile_body,
        in_specs=[pl.BlockSpec(memory_space=pl.ANY)],
        out_specs=pl.BlockSpec(memory_space=pl.ANY),
        out_shape=jax.ShapeDtypeStruct((FLAT,), jnp.float32),
        scratch_shapes=[
            pltpu.VMEM((2, 8, DETILE_BLK), jnp.float32),
            pltpu.SemaphoreType.DMA((2,)),
            pltpu.SemaphoreType.DMA((2,)),
        ],
    )(table_t)


@jax.jit
def kernel(u, v, embed):
    mesh = plsc.VectorSubcoreMesh(
        core_axis_name="c", subcore_axis_name="s",
        num_cores=NC, num_subcores=NS,
    )
    k = pl.kernel(
        _sc_body,
        out_type=jax.ShapeDtypeStruct((BATCH,), jnp.float32),
        mesh=mesh,
        scratch_types=[
            pltpu.VMEM((N_CHUNKS, CHUNK), jnp.int32),
            pltpu.VMEM((N_CHUNKS, CHUNK), jnp.int32),
            pltpu.VMEM((2, DIM, CHUNK), jnp.int32),
            pltpu.VMEM((2, DIM, CHUNK), jnp.int32),
            pltpu.VMEM((2, DIM, CHUNK), jnp.float32),
            pltpu.VMEM((2, DIM, CHUNK), jnp.float32),
            pltpu.VMEM((DIM * VTAIL,), jnp.float32),
            pltpu.VMEM((B_PER_W,), jnp.float32),
            pltpu.SemaphoreType.DMA,
        ],
        compiler_params=pltpu.CompilerParams(
            needs_layout_passes=False, use_tc_tiling_on_sc=False),
    )
    tflat = _detile(embed.T)
    tail = jnp.reshape(embed[VBULK:].T, (DIM * VTAIL,))
    return k(u.astype(jnp.int32), v.astype(jnp.int32), tflat, tail)


# speedup vs baseline: 13.4821x; 1.0867x over previous
import jax
import jax.numpy as jnp
from jax import lax
from jax.experimental import pallas as pl
from jax.experimental.pallas import tpu as pltpu
from jax.experimental.pallas import tpu_sc as plsc

VOCAB = 1000000
DIM = 16
BATCH = 16384

NC = 2
NS = 16
NW = NC * NS
LANES = 16

B_PER_W = BATCH // NW
CHUNK = 128
N_CHUNKS = B_PER_W // CHUNK
GROUPS = CHUNK // LANES

VPAD = 1048576
VBULK = 999936
VTAIL = VOCAB - VBULK
DETILE_BLK = 262144
FLAT = DIM * VPAD


def _sc_body(u_hbm, v_hbm, table_hbm, tail_hbm, out_hbm,
             idx_u, idx_v, off_u, off_v, ubuf, vbuf, tail_v, out_loc, sem):
    wid = lax.axis_index("s") * NC + lax.axis_index("c")
    base = wid * B_PER_W
    pltpu.sync_copy(tail_hbm, tail_v)
    for c in range(N_CHUNKS):
        pltpu.sync_copy(u_hbm.at[pl.ds(base + c * CHUNK, CHUNK)], idx_u.at[c])
        pltpu.sync_copy(v_hbm.at[pl.ds(base + c * CHUNK, CHUNK)], idx_v.at[c])

    def build_offsets(c, b):
        def slice_fn(g, _):
            o = g * LANES
            ru = idx_u.at[c][pl.ds(o, LANES)]
            rv = idx_v.at[c][pl.ds(o, LANES)]
            ru = jnp.minimum(ru, VBULK - 1)
            rv = jnp.minimum(rv, VBULK - 1)
            for d in range(DIM):
                off_u[b, d, pl.ds(o, LANES)] = ru + d * VPAD
                off_v[b, d, pl.ds(o, LANES)] = rv + d * VPAD
            return _

        lax.fori_loop(0, GROUPS, slice_fn, None)

    def fire(b):
        copies = []
        for d in range(DIM):
            copies.append(pltpu.async_copy(
                table_hbm.at[off_u.at[b, d]], ubuf.at[b, d], sem))
            copies.append(pltpu.async_copy(
                table_hbm.at[off_v.at[b, d]], vbuf.at[b, d], sem))
        return copies

    def compute_chunk(c, b):
        def group(g, _):
            o = g * LANES
            ru = idx_u.at[c][pl.ds(o, LANES)]
            rv = idx_v.at[c][pl.ds(o, LANES)]
            mu = ru >= VBULK
            mv = rv >= VBULK
            tu = jnp.clip(ru - VBULK, 0, VTAIL - 1)
            tv = jnp.clip(rv - VBULK, 0, VTAIL - 1)
            acc = jnp.zeros((LANES,), jnp.float32)
            for d in range(DIM):
                uu = ubuf.at[b, d][pl.ds(o, LANES)]
                vv = vbuf.at[b, d][pl.ds(o, LANES)]
                uu = jnp.where(mu, plsc.load_gather(tail_v, [tu + d * VTAIL]), uu)
                vv = jnp.where(mv, plsc.load_gather(tail_v, [tv + d * VTAIL]), vv)
                acc = acc + uu * vv
            out_loc[pl.ds(c * CHUNK + o, LANES)] = 1.0 / (1.0 + jnp.exp(-acc))
            return _

        lax.fori_loop(0, GROUPS, group, None)

    build_offsets(0, 0)
    pending = fire(0)
    for c in range(N_CHUNKS):
        b = c % 2
        if c + 1 < N_CHUNKS:
            build_offsets(c + 1, 1 - b)
        for cp in pending:
            cp.wait()
        if c + 1 < N_CHUNKS:
            nxt = fire(1 - b)
        compute_chunk(c, b)
        if c + 1 < N_CHUNKS:
            pending = nxt

    pltpu.sync_copy(out_loc, out_hbm.at[pl.ds(base, B_PER_W)])


def _detile_body(in_hbm, out_hbm, buf, sem_in, sem_out):
    steps = []
    for p in range(DIM // 8):
        for k in range((VBULK + DETILE_BLK - 1) // DETILE_BLK):
            n = min(DETILE_BLK, VBULK - k * DETILE_BLK)
            steps.append((p, k, n))

    def fire_in(s, b):
        p, k, n = steps[s]
        cp = pltpu.make_async_copy(
            in_hbm.at[pl.ds(p * 8, 8), pl.ds(k * DETILE_BLK, n)],
            buf.at[b, :, pl.ds(0, n)], sem_in.at[b])
        cp.start()
        return cp

    def fire_out(s, b):
        p, k, n = steps[s]
        cps = []
        for j in range(8):
            cp = pltpu.make_async_copy(
                buf.at[b, j, pl.ds(0, n)],
                out_hbm.at[pl.ds((p * 8 + j) * VPAD + k * DETILE_BLK, n)],
                sem_out.at[b])
            cp.start()
            cps.append(cp)
        return cps

    nsteps = len(steps)
    pending_in = fire_in(0, 0)
    pending_out = None
    for s in range(nsteps):
        b = s % 2
        pending_in.wait()
        if s + 1 < nsteps:
            if pending_out is not None:
                for cp in pending_out:
                    cp.wait()
            nxt = fire_in(s + 1, 1 - b)
        out_cps = fire_out(s, b)
        if s + 1 < nsteps:
            pending_out, pending_in = out_cps, nxt
        else:
            if pending_out is not None:
                for cp in pending_out:
                    cp.wait()
            for cp in out_cps:
                cp.wait()


def _detile(table_t):
    return pl.pallas_call(
        _detile_body,
        in_specs=[pl.BlockSpec(memory_space=pl.ANY)],
        out_specs=pl.BlockSpec(memory_space=pl.ANY),
        out_shape=jax.ShapeDtypeStruct((FLAT,), jnp.float32),
        scratch_shapes=[
            pltpu.VMEM((2, 8, DETILE_BLK), jnp.float32),
            pltpu.SemaphoreType.DMA((2,)),
            pltpu.SemaphoreType.DMA((2,)),
        ],
    )(table_t)


@jax.jit
def kernel(u, v, embed):
    mesh = plsc.VectorSubcoreMesh(
        core_axis_name="c", subcore_axis_name="s",
        num_cores=NC, num_subcores=NS,
    )
    k = pl.kernel(
        _sc_body,
        out_type=jax.ShapeDtypeStruct((BATCH,), jnp.float32),
        mesh=mesh,
        scratch_types=[
            pltpu.VMEM((N_CHUNKS, CHUNK), jnp.int32),
            pltpu.VMEM((N_CHUNKS, CHUNK), jnp.int32),
            pltpu.VMEM((2, DIM, CHUNK), jnp.int32),
            pltpu.VMEM((2, DIM, CHUNK), jnp.int32),
            pltpu.VMEM((2, DIM, CHUNK), jnp.float32),
            pltpu.VMEM((2, DIM, CHUNK), jnp.float32),
            pltpu.VMEM((DIM * VTAIL,), jnp.float32),
            pltpu.VMEM((B_PER_W,), jnp.float32),
            pltpu.SemaphoreType.DMA,
        ],
        compiler_params=pltpu.CompilerParams(
            needs_layout_passes=False, use_tc_tiling_on_sc=False),
    )
    tflat = _detile(embed.T)
    tail = jnp.reshape(embed[VBULK:].T, (DIM * VTAIL,))
    return k(u.astype(jnp.int32), v.astype(jnp.int32), tflat, tail)


# speedup vs baseline: 14.2628x; 1.0579x over previous
import jax
import jax.numpy as jnp
from jax import lax
from jax.experimental import pallas as pl
from jax.experimental.pallas import tpu as pltpu
from jax.experimental.pallas import tpu_sc as plsc

VOCAB = 1000000
DIM = 16
BATCH = 16384

NC = 2
NS = 16
NW = NC * NS
LANES = 16

B_PER_W = BATCH // NW
CHUNK = 128
N_CHUNKS = B_PER_W // CHUNK
GROUPS = CHUNK // LANES

VPAD = 1048576
VBULK = 999936
VTAIL = VOCAB - VBULK
DETILE_BLK = 524288
FLAT = DIM * VPAD


def _sc_body(u_hbm, v_hbm, table_hbm, tail_hbm, out_hbm,
             idx_u, idx_v, off_u, off_v, ubuf, vbuf, tail_v, out_loc, sem):
    wid = lax.axis_index("s") * NC + lax.axis_index("c")
    base = wid * B_PER_W
    pltpu.sync_copy(tail_hbm, tail_v)
    for c in range(N_CHUNKS):
        pltpu.sync_copy(u_hbm.at[pl.ds(base + c * CHUNK, CHUNK)], idx_u.at[c])
        pltpu.sync_copy(v_hbm.at[pl.ds(base + c * CHUNK, CHUNK)], idx_v.at[c])

    def build_offsets(c, b):
        def slice_fn(g, _):
            o = g * LANES
            ru = idx_u.at[c][pl.ds(o, LANES)]
            rv = idx_v.at[c][pl.ds(o, LANES)]
            ru = jnp.minimum(ru, VBULK - 1)
            rv = jnp.minimum(rv, VBULK - 1)
            for d in range(DIM):
                off_u[b, d, pl.ds(o, LANES)] = ru + d * VPAD
                off_v[b, d, pl.ds(o, LANES)] = rv + d * VPAD
            return _

        lax.fori_loop(0, GROUPS, slice_fn, None)

    def fire(b):
        copies = []
        for d in range(DIM):
            copies.append(pltpu.async_copy(
                table_hbm.at[off_u.at[b, d]], ubuf.at[b, d], sem))
            copies.append(pltpu.async_copy(
                table_hbm.at[off_v.at[b, d]], vbuf.at[b, d], sem))
        return copies

    def compute_chunk(c, b):
        def group(g, _):
            o = g * LANES
            ru = idx_u.at[c][pl.ds(o, LANES)]
            rv = idx_v.at[c][pl.ds(o, LANES)]
            mu = ru >= VBULK
            mv = rv >= VBULK
            tu = jnp.clip(ru - VBULK, 0, VTAIL - 1)
            tv = jnp.clip(rv - VBULK, 0, VTAIL - 1)
            acc = jnp.zeros((LANES,), jnp.float32)
            for d in range(DIM):
                uu = ubuf.at[b, d][pl.ds(o, LANES)]
                vv = vbuf.at[b, d][pl.ds(o, LANES)]
                uu = jnp.where(mu, plsc.load_gather(tail_v, [tu + d * VTAIL]), uu)
                vv = jnp.where(mv, plsc.load_gather(tail_v, [tv + d * VTAIL]), vv)
                acc = acc + uu * vv
            out_loc[pl.ds(c * CHUNK + o, LANES)] = 1.0 / (1.0 + jnp.exp(-acc))
            return _

        lax.fori_loop(0, GROUPS, group, None)

    build_offsets(0, 0)
    pending = fire(0)
    for c in range(N_CHUNKS):
        b = c % 2
        if c + 1 < N_CHUNKS:
            build_offsets(c + 1, 1 - b)
        for cp in pending:
            cp.wait()
        if c + 1 < N_CHUNKS:
            nxt = fire(1 - b)
        compute_chunk(c, b)
        if c + 1 < N_CHUNKS:
            pending = nxt

    pltpu.sync_copy(out_loc, out_hbm.at[pl.ds(base, B_PER_W)])


def _detile_body(in_hbm, out_hbm, buf, sem_in, sem_out):
    steps = []
    for p in range(DIM // 8):
        for k in range((VBULK + DETILE_BLK - 1) // DETILE_BLK):
            n = min(DETILE_BLK, VBULK - k * DETILE_BLK)
            steps.append((p, k, n))

    def fire_in(s, b):
        p, k, n = steps[s]
        cp = pltpu.make_async_copy(
            in_hbm.at[pl.ds(p * 8, 8), pl.ds(k * DETILE_BLK, n)],
            buf.at[b, :, pl.ds(0, n)], sem_in.at[b])
        cp.start()
        return cp

    def fire_out(s, b):
        p, k, n = steps[s]
        cps = []
        for j in range(8):
            cp = pltpu.make_async_copy(
                buf.at[b, j, pl.ds(0, n)],
                out_hbm.at[pl.ds((p * 8 + j) * VPAD + k * DETILE_BLK, n)],
                sem_out.at[b])
            cp.start()
            cps.append(cp)
        return cps

    nsteps = len(steps)
    pending_in = fire_in(0, 0)
    pending_out = None
    for s in range(nsteps):
        b = s % 2
        pending_in.wait()
        if s + 1 < nsteps:
            if pending_out is not None:
                for cp in pending_out:
                    cp.wait()
            nxt = fire_in(s + 1, 1 - b)
        out_cps = fire_out(s, b)
        if s + 1 < nsteps:
            pending_out, pending_in = out_cps, nxt
        else:
            if pending_out is not None:
                for cp in pending_out:
                    cp.wait()
            for cp in out_cps:
                cp.wait()


def _detile(table_t):
    return pl.pallas_call(
        _detile_body,
        in_specs=[pl.BlockSpec(memory_space=pl.ANY)],
        out_specs=pl.BlockSpec(memory_space=pl.ANY),
        out_shape=jax.ShapeDtypeStruct((FLAT,), jnp.float32),
        scratch_shapes=[
            pltpu.VMEM((2, 8, DETILE_BLK), jnp.float32),
            pltpu.SemaphoreType.DMA((2,)),
            pltpu.SemaphoreType.DMA((2,)),
        ],
    )(table_t)


@jax.jit
def kernel(u, v, embed):
    mesh = plsc.VectorSubcoreMesh(
        core_axis_name="c", subcore_axis_name="s",
        num_cores=NC, num_subcores=NS,
    )
    k = pl.kernel(
        _sc_body,
        out_type=jax.ShapeDtypeStruct((BATCH,), jnp.float32),
        mesh=mesh,
        scratch_types=[
            pltpu.VMEM((N_CHUNKS, CHUNK), jnp.int32),
            pltpu.VMEM((N_CHUNKS, CHUNK), jnp.int32),
            pltpu.VMEM((2, DIM, CHUNK), jnp.int32),
            pltpu.VMEM((2, DIM, CHUNK), jnp.int32),
            pltpu.VMEM((2, DIM, CHUNK), jnp.float32),
            pltpu.VMEM((2, DIM, CHUNK), jnp.float32),
            pltpu.VMEM((DIM * VTAIL,), jnp.float32),
            pltpu.VMEM((B_PER_W,), jnp.float32),
            pltpu.SemaphoreType.DMA,
        ],
        compiler_params=pltpu.CompilerParams(
            needs_layout_passes=False, use_tc_tiling_on_sc=False),
    )
    tflat = _detile(embed.T)
    tail = jnp.reshape(embed[VBULK:].T, (DIM * VTAIL,))
    return k(u.astype(jnp.int32), v.astype(jnp.int32), tflat, tail)


# speedup vs baseline: 14.7166x; 1.0318x over previous
import jax
import jax.numpy as jnp
from jax import lax
from jax.experimental import pallas as pl
from jax.experimental.pallas import tpu as pltpu
from jax.experimental.pallas import tpu_sc as plsc

VOCAB = 1000000
DIM = 16
BATCH = 16384

NC = 2
NS = 16
NW = NC * NS
LANES = 16

B_PER_W = BATCH // NW
CHUNK = 128
N_CHUNKS = B_PER_W // CHUNK
GROUPS = CHUNK // LANES

VPAD = 1048576
VBULK = 999936
VTAIL = VOCAB - VBULK
DETILE_BLK = 786432
FLAT = DIM * VPAD


def _sc_body(u_hbm, v_hbm, table_hbm, tail_hbm, out_hbm,
             idx_u, idx_v, off_u, off_v, ubuf, vbuf, tail_v, out_loc, sem):
    wid = lax.axis_index("s") * NC + lax.axis_index("c")
    base = wid * B_PER_W
    pltpu.sync_copy(tail_hbm, tail_v)
    for c in range(N_CHUNKS):
        pltpu.sync_copy(u_hbm.at[pl.ds(base + c * CHUNK, CHUNK)], idx_u.at[c])
        pltpu.sync_copy(v_hbm.at[pl.ds(base + c * CHUNK, CHUNK)], idx_v.at[c])

    def build_offsets(c, b):
        def slice_fn(g, _):
            o = g * LANES
            ru = idx_u.at[c][pl.ds(o, LANES)]
            rv = idx_v.at[c][pl.ds(o, LANES)]
            ru = jnp.minimum(ru, VBULK - 1)
            rv = jnp.minimum(rv, VBULK - 1)
            for d in range(DIM):
                off_u[b, d, pl.ds(o, LANES)] = ru + d * VPAD
                off_v[b, d, pl.ds(o, LANES)] = rv + d * VPAD
            return _

        lax.fori_loop(0, GROUPS, slice_fn, None)

    def fire(b):
        copies = []
        for d in range(DIM):
            copies.append(pltpu.async_copy(
                table_hbm.at[off_u.at[b, d]], ubuf.at[b, d], sem))
            copies.append(pltpu.async_copy(
                table_hbm.at[off_v.at[b, d]], vbuf.at[b, d], sem))
        return copies

    def compute_chunk(c, b):
        def group(g, _):
            o = g * LANES
            ru = idx_u.at[c][pl.ds(o, LANES)]
            rv = idx_v.at[c][pl.ds(o, LANES)]
            mu = ru >= VBULK
            mv = rv >= VBULK
            tu = jnp.clip(ru - VBULK, 0, VTAIL - 1)
            tv = jnp.clip(rv - VBULK, 0, VTAIL - 1)
            acc = jnp.zeros((LANES,), jnp.float32)
            for d in range(DIM):
                uu = ubuf.at[b, d][pl.ds(o, LANES)]
                vv = vbuf.at[b, d][pl.ds(o, LANES)]
                uu = jnp.where(mu, plsc.load_gather(tail_v, [tu + d * VTAIL]), uu)
                vv = jnp.where(mv, plsc.load_gather(tail_v, [tv + d * VTAIL]), vv)
                acc = acc + uu * vv
            out_loc[pl.ds(c * CHUNK + o, LANES)] = 1.0 / (1.0 + jnp.exp(-acc))
            return _

        lax.fori_loop(0, GROUPS, group, None)

    build_offsets(0, 0)
    pending = fire(0)
    for c in range(N_CHUNKS):
        b = c % 2
        if c + 1 < N_CHUNKS:
            build_offsets(c + 1, 1 - b)
        for cp in pending:
            cp.wait()
        if c + 1 < N_CHUNKS:
            nxt = fire(1 - b)
        compute_chunk(c, b)
        if c + 1 < N_CHUNKS:
            pending = nxt

    pltpu.sync_copy(out_loc, out_hbm.at[pl.ds(base, B_PER_W)])


def _detile_body(in_hbm, out_hbm, buf, sem_in, sem_out):
    steps = []
    for p in range(DIM // 8):
        for k in range((VBULK + DETILE_BLK - 1) // DETILE_BLK):
            n = min(DETILE_BLK, VBULK - k * DETILE_BLK)
            steps.append((p, k, n))

    def fire_in(s, b):
        p, k, n = steps[s]
        cp = pltpu.make_async_copy(
            in_hbm.at[pl.ds(p * 8, 8), pl.ds(k * DETILE_BLK, n)],
            buf.at[b, :, pl.ds(0, n)], sem_in.at[b])
        cp.start()
        return cp

    def fire_out(s, b):
        p, k, n = steps[s]
        cps = []
        for j in range(8):
            cp = pltpu.make_async_copy(
                buf.at[b, j, pl.ds(0, n)],
                out_hbm.at[pl.ds((p * 8 + j) * VPAD + k * DETILE_BLK, n)],
                sem_out.at[b])
            cp.start()
            cps.append(cp)
        return cps

    nsteps = len(steps)
    pending_in = fire_in(0, 0)
    pending_out = None
    for s in range(nsteps):
        b = s % 2
        pending_in.wait()
        if s + 1 < nsteps:
            if pending_out is not None:
                for cp in pending_out:
                    cp.wait()
            nxt = fire_in(s + 1, 1 - b)
        out_cps = fire_out(s, b)
        if s + 1 < nsteps:
            pending_out, pending_in = out_cps, nxt
        else:
            if pending_out is not None:
                for cp in pending_out:
                    cp.wait()
            for cp in out_cps:
                cp.wait()


def _detile(table_t):
    return pl.pallas_call(
        _detile_body,
        in_specs=[pl.BlockSpec(memory_space=pl.ANY)],
        out_specs=pl.BlockSpec(memory_space=pl.ANY),
        out_shape=jax.ShapeDtypeStruct((FLAT,), jnp.float32),
        scratch_shapes=[
            pltpu.VMEM((2, 8, DETILE_BLK), jnp.float32),
            pltpu.SemaphoreType.DMA((2,)),
            pltpu.SemaphoreType.DMA((2,)),
        ],
    )(table_t)


@jax.jit
def kernel(u, v, embed):
    mesh = plsc.VectorSubcoreMesh(
        core_axis_name="c", subcore_axis_name="s",
        num_cores=NC, num_subcores=NS,
    )
    k = pl.kernel(
        _sc_body,
        out_type=jax.ShapeDtypeStruct((BATCH,), jnp.float32),
        mesh=mesh,
        scratch_types=[
            pltpu.VMEM((N_CHUNKS, CHUNK), jnp.int32),
            pltpu.VMEM((N_CHUNKS, CHUNK), jnp.int32),
            pltpu.VMEM((2, DIM, CHUNK), jnp.int32),
            pltpu.VMEM((2, DIM, CHUNK), jnp.int32),
            pltpu.VMEM((2, DIM, CHUNK), jnp.float32),
            pltpu.VMEM((2, DIM, CHUNK), jnp.float32),
            pltpu.VMEM((DIM * VTAIL,), jnp.float32),
            pltpu.VMEM((B_PER_W,), jnp.float32),
            pltpu.SemaphoreType.DMA,
        ],
        compiler_params=pltpu.CompilerParams(
            needs_layout_passes=False, use_tc_tiling_on_sc=False),
    )
    tflat = _detile(embed.T)
    tail = jnp.reshape(embed[VBULK:].T, (DIM * VTAIL,))
    return k(u.astype(jnp.int32), v.astype(jnp.int32), tflat, tail)
